# Initial kernel scaffold; baseline (speedup 1.0000x reference)
import functools

import jax
import jax.numpy as jnp
from jax import lax
from jax.experimental import pallas as pl
from jax.experimental.pallas import tpu as pltpu
from jax.experimental.pallas import tpu_sc as plsc

N_NODES = 10000
N_EDGES = 320000
D_IN = 128
D_HID = 64
N_HID = 6

NC = 2
NS = 16
NW = NC * NS
EPT = N_EDGES // NW
CHUNK = 80
NCHUNK = EPT // CHUNK
RPT = N_NODES // NS
BR = 1000


@functools.lru_cache(maxsize=None)
def _make_sc_aggregate(width: int):
    mesh = plsc.VectorSubcoreMesh(core_axis_name="c", subcore_axis_name="s")

    def body(table, src_i, dst_i, out, tab_s, acc_s, stage_v, src_v, dst_v,
             rows_v, sem):
        c = lax.axis_index("c")
        s = lax.axis_index("s")
        wid = c * NS + s
        r0 = s * RPT
        pltpu.sync_copy(table.at[pl.ds(r0, RPT)], stage_v)
        pltpu.sync_copy(stage_v, tab_s.at[pl.ds(r0, RPT)])
        pltpu.sync_copy(stage_v, acc_s.at[pl.ds(r0, RPT)])
        pltpu.sync_copy(src_i.at[wid], src_v)
        pltpu.sync_copy(dst_i.at[wid], dst_v)
        plsc.subcore_barrier()

        def step(j, carry):
            pltpu.async_copy(tab_s.at[src_v.at[j]], rows_v, sem).wait()
            pltpu.sync_copy(rows_v, acc_s.at[dst_v.at[j]], add=True)
            return carry

        lax.fori_loop(0, NCHUNK, step, 0)
        plsc.subcore_barrier()
        pltpu.sync_copy(acc_s.at[pl.ds(r0, RPT)], stage_v)
        pltpu.sync_copy(stage_v, out.at[c, pl.ds(r0, RPT)])

    return pl.kernel(
        body,
        out_type=jax.ShapeDtypeStruct((NC, N_NODES, width), jnp.float32),
        mesh=mesh,
        scratch_types=[
            pltpu.VMEM_SHARED((N_NODES, width), jnp.float32),
            pltpu.VMEM_SHARED((N_NODES, width), jnp.float32),
            pltpu.VMEM((RPT, width), jnp.float32),
            pltpu.VMEM((NCHUNK, CHUNK), jnp.int32),
            pltpu.VMEM((NCHUNK, CHUNK), jnp.int32),
            pltpu.VMEM((CHUNK, width), jnp.float32),
            pltpu.SemaphoreType.DMA,
        ],
    )


def _tc_prelude(degp, x, w):
    def body(p_ref, x_ref, w_ref, hs_ref, dinv_ref):
        deg = p_ref[0, :, 0:1] + p_ref[1, :, 0:1] - 1.0
        dinv = lax.rsqrt(deg)
        h = jnp.dot(x_ref[...], w_ref[...], preferred_element_type=jnp.float32)
        hs_ref[...] = h * dinv
        dinv_ref[...] = dinv

    return pl.pallas_call(
        body,
        grid=(N_NODES // BR,),
        in_specs=[
            pl.BlockSpec((NC, BR, 16), lambda i: (0, i, 0)),
            pl.BlockSpec((BR, D_IN), lambda i: (i, 0)),
            pl.BlockSpec((D_IN, D_HID), lambda i: (0, 0)),
        ],
        out_specs=[
            pl.BlockSpec((BR, D_HID), lambda i: (i, 0)),
            pl.BlockSpec((BR, 1), lambda i: (i, 0)),
        ],
        out_shape=[
            jax.ShapeDtypeStruct((N_NODES, D_HID), jnp.float32),
            jax.ShapeDtypeStruct((N_NODES, 1), jnp.float32),
        ],
    )(degp, x, w)


def _tc_layer(p, hs, dinv, w, b):
    wout = w.shape[1]

    def body(p_ref, hs_ref, dinv_ref, w_ref, b_ref, out_ref):
        agg = p_ref[0] + p_ref[1] - hs_ref[...]
        t = agg * dinv_ref[...] + b_ref[...]
        r = jnp.maximum(t, 0.0)
        out_ref[...] = (
            jnp.dot(r, w_ref[...], preferred_element_type=jnp.float32)
            * dinv_ref[...]
        )

    return pl.pallas_call(
        body,
        grid=(N_NODES // BR,),
        in_specs=[
            pl.BlockSpec((NC, BR, D_HID), lambda i: (0, i, 0)),
            pl.BlockSpec((BR, D_HID), lambda i: (i, 0)),
            pl.BlockSpec((BR, 1), lambda i: (i, 0)),
            pl.BlockSpec((D_HID, wout), lambda i: (0, 0)),
            pl.BlockSpec((1, D_HID), lambda i: (0, 0)),
        ],
        out_specs=pl.BlockSpec((BR, wout), lambda i: (i, 0)),
        out_shape=jax.ShapeDtypeStruct((N_NODES, wout), jnp.float32),
    )(p, hs, dinv, w, b)


def _tc_final(p, z, dinv, b):
    def body(p_ref, z_ref, dinv_ref, b_ref, out_ref):
        agg = p_ref[0] + p_ref[1] - z_ref[...]
        out_ref[...] = agg[:, 0:1] * dinv_ref[...] + b_ref[...]

    return pl.pallas_call(
        body,
        grid=(N_NODES // BR,),
        in_specs=[
            pl.BlockSpec((NC, BR, 16), lambda i: (0, i, 0)),
            pl.BlockSpec((BR, 16), lambda i: (i, 0)),
            pl.BlockSpec((BR, 1), lambda i: (i, 0)),
            pl.BlockSpec((1, 1), lambda i: (0, 0)),
        ],
        out_specs=pl.BlockSpec((BR, 1), lambda i: (i, 0)),
        out_shape=jax.ShapeDtypeStruct((N_NODES, 1), jnp.float32),
    )(p, z, dinv, b)


def kernel(x, edge_index, W_i, b_i, W_h, b_h, W_o, b_o):
    ei = edge_index.astype(jnp.int32)
    src3 = ei[0].reshape(NW, NCHUNK, CHUNK)
    dst3 = ei[1].reshape(NW, NCHUNK, CHUNK)
    onehot = jnp.zeros((N_NODES, 16), jnp.float32).at[:, 0].set(1.0)

    agg16 = _make_sc_aggregate(16)
    agg64 = _make_sc_aggregate(D_HID)

    degp = agg16(onehot, src3, dst3)
    hs, dinv = _tc_prelude(degp, x, W_i)

    w_o16 = jnp.pad(W_o, ((0, 0), (0, 15)))
    weights = [W_h[i] for i in range(N_HID)] + [w_o16]
    biases = [b_i] + [b_h[i] for i in range(N_HID)]

    p = agg64(hs, src3, dst3)
    for i in range(N_HID + 1):
        nxt = _tc_layer(p, hs, dinv, weights[i], biases[i].reshape(1, D_HID))
        hs = nxt
        p = (agg64 if i < N_HID else agg16)(nxt, src3, dst3)
    return _tc_final(p, hs, dinv, b_o.reshape(1, 1))


# speedup vs baseline: 20.6566x; 20.6566x over previous
import functools

import jax
import jax.numpy as jnp
from jax import lax
from jax.experimental import pallas as pl
from jax.experimental.pallas import tpu as pltpu
from jax.experimental.pallas import tpu_sc as plsc

N_NODES = 10000
N_EDGES = 320000
D_IN = 128
D_HID = 64
N_HID = 6

NC = 2
NS = 16
NW = NC * NS
EPT = N_EDGES // NW
CHUNK = 80
NCHUNK = EPT // CHUNK
NOUTER = 5
NINNER = NCHUNK // NOUTER
RPT = 624
TAIL = N_NODES - NS * RPT
TAIL0 = NS * RPT
BR = 1000


@functools.lru_cache(maxsize=None)
def _make_sc_aggregate(width: int):
    mesh = plsc.VectorSubcoreMesh(core_axis_name="c", subcore_axis_name="s")

    def body(table, src_i, dst_i, out, tab_s, acc_s, src_v, dst_v, rows_v,
             sem):
        c = lax.axis_index("c")
        s = lax.axis_index("s")
        wid = c * NS + s
        r0 = s * RPT
        pltpu.sync_copy(table.at[pl.ds(r0, RPT)], tab_s.at[pl.ds(r0, RPT)])
        pltpu.sync_copy(table.at[pl.ds(r0, RPT)], acc_s.at[pl.ds(r0, RPT)])

        @pl.when(s == 0)
        def _init_tail():
            pltpu.sync_copy(table.at[pl.ds(TAIL0, TAIL)],
                            tab_s.at[pl.ds(TAIL0, TAIL)])
            pltpu.sync_copy(table.at[pl.ds(TAIL0, TAIL)],
                            acc_s.at[pl.ds(TAIL0, TAIL)])

        plsc.subcore_barrier()

        def outer(o, carry):
            pltpu.sync_copy(src_i.at[wid, o], src_v)
            pltpu.sync_copy(dst_i.at[wid, o], dst_v)

            def step(j, carry2):
                pltpu.async_copy(tab_s.at[src_v.at[j]], rows_v, sem).wait()
                pltpu.sync_copy(rows_v, acc_s.at[dst_v.at[j]], add=True)
                return carry2

            return lax.fori_loop(0, NINNER, step, carry)

        lax.fori_loop(0, NOUTER, outer, 0)
        plsc.subcore_barrier()
        pltpu.sync_copy(acc_s.at[pl.ds(r0, RPT)], out.at[c, pl.ds(r0, RPT)])

        @pl.when(s == 0)
        def _out_tail():
            pltpu.sync_copy(acc_s.at[pl.ds(TAIL0, TAIL)],
                            out.at[c, pl.ds(TAIL0, TAIL)])

    return pl.kernel(
        body,
        out_type=jax.ShapeDtypeStruct((NC, N_NODES, width), jnp.float32),
        mesh=mesh,
        scratch_types=[
            pltpu.VMEM_SHARED((N_NODES, width), jnp.float32),
            pltpu.VMEM_SHARED((N_NODES, width), jnp.float32),
            pltpu.VMEM((NINNER, CHUNK), jnp.int32),
            pltpu.VMEM((NINNER, CHUNK), jnp.int32),
            pltpu.VMEM((CHUNK, width), jnp.float32),
            pltpu.SemaphoreType.DMA,
        ],
    )


def _tc_prelude(degp, x, w):
    def body(p_ref, x_ref, w_ref, hs_ref, dinv_ref):
        deg = p_ref[0, :, 0:1] + p_ref[1, :, 0:1] - 1.0
        dinv = lax.rsqrt(deg)
        h = jnp.dot(x_ref[...], w_ref[...], preferred_element_type=jnp.float32)
        hs_ref[...] = h * dinv
        dinv_ref[...] = dinv

    return pl.pallas_call(
        body,
        grid=(N_NODES // BR,),
        in_specs=[
            pl.BlockSpec((NC, BR, 16), lambda i: (0, i, 0)),
            pl.BlockSpec((BR, D_IN), lambda i: (i, 0)),
            pl.BlockSpec((D_IN, D_HID), lambda i: (0, 0)),
        ],
        out_specs=[
            pl.BlockSpec((BR, D_HID), lambda i: (i, 0)),
            pl.BlockSpec((BR, 1), lambda i: (i, 0)),
        ],
        out_shape=[
            jax.ShapeDtypeStruct((N_NODES, D_HID), jnp.float32),
            jax.ShapeDtypeStruct((N_NODES, 1), jnp.float32),
        ],
    )(degp, x, w)


def _tc_layer(p, hs, dinv, w, b):
    wout = w.shape[1]

    def body(p_ref, hs_ref, dinv_ref, w_ref, b_ref, out_ref):
        agg = p_ref[0] + p_ref[1] - hs_ref[...]
        t = agg * dinv_ref[...] + b_ref[...]
        r = jnp.maximum(t, 0.0)
        out_ref[...] = (
            jnp.dot(r, w_ref[...], preferred_element_type=jnp.float32)
            * dinv_ref[...]
        )

    return pl.pallas_call(
        body,
        grid=(N_NODES // BR,),
        in_specs=[
            pl.BlockSpec((NC, BR, D_HID), lambda i: (0, i, 0)),
            pl.BlockSpec((BR, D_HID), lambda i: (i, 0)),
            pl.BlockSpec((BR, 1), lambda i: (i, 0)),
            pl.BlockSpec((D_HID, wout), lambda i: (0, 0)),
            pl.BlockSpec((1, D_HID), lambda i: (0, 0)),
        ],
        out_specs=pl.BlockSpec((BR, wout), lambda i: (i, 0)),
        out_shape=jax.ShapeDtypeStruct((N_NODES, wout), jnp.float32),
    )(p, hs, dinv, w, b)


def _tc_final(p, z, dinv, b):
    def body(p_ref, z_ref, dinv_ref, b_ref, out_ref):
        agg = p_ref[0] + p_ref[1] - z_ref[...]
        out_ref[...] = agg[:, 0:1] * dinv_ref[...] + b_ref[...]

    return pl.pallas_call(
        body,
        grid=(N_NODES // BR,),
        in_specs=[
            pl.BlockSpec((NC, BR, 16), lambda i: (0, i, 0)),
            pl.BlockSpec((BR, 16), lambda i: (i, 0)),
            pl.BlockSpec((BR, 1), lambda i: (i, 0)),
            pl.BlockSpec((1, 1), lambda i: (0, 0)),
        ],
        out_specs=pl.BlockSpec((BR, 1), lambda i: (i, 0)),
        out_shape=jax.ShapeDtypeStruct((N_NODES, 1), jnp.float32),
    )(p, z, dinv, b)


def kernel(x, edge_index, W_i, b_i, W_h, b_h, W_o, b_o):
    ei = edge_index.astype(jnp.int32)
    src3 = ei[0].reshape(NW, NOUTER, NINNER, CHUNK)
    dst3 = ei[1].reshape(NW, NOUTER, NINNER, CHUNK)
    onehot = jnp.zeros((N_NODES, 16), jnp.float32).at[:, 0].set(1.0)

    agg16 = _make_sc_aggregate(16)
    agg64 = _make_sc_aggregate(D_HID)

    degp = agg16(onehot, src3, dst3)
    hs, dinv = _tc_prelude(degp, x, W_i)

    w_o16 = jnp.pad(W_o, ((0, 0), (0, 15)))
    weights = [W_h[i] for i in range(N_HID)] + [w_o16]
    biases = [b_i] + [b_h[i] for i in range(N_HID)]

    p = agg64(hs, src3, dst3)
    for i in range(N_HID + 1):
        nxt = _tc_layer(p, hs, dinv, weights[i], biases[i].reshape(1, D_HID))
        hs = nxt
        p = (agg64 if i < N_HID else agg16)(nxt, src3, dst3)
    return _tc_final(p, hs, dinv, b_o.reshape(1, 1))


# speedup vs baseline: 23.4874x; 1.1370x over previous
import functools

import jax
import jax.numpy as jnp
from jax import lax
from jax.experimental import pallas as pl
from jax.experimental.pallas import tpu as pltpu
from jax.experimental.pallas import tpu_sc as plsc

N_NODES = 10000
N_EDGES = 320000
D_IN = 128
D_HID = 64
N_HID = 6

NC = 2
NS = 16
EPT = N_EDGES // NS
CHUNK = 80
NCHUNK = EPT // CHUNK
NOUTER = 10
NINNER = NCHUNK // NOUTER
NBUF = 5
NROUND = NINNER // NBUF
RPT = 624
TAIL = N_NODES - NS * RPT
TAIL0 = NS * RPT
BR = 1000


@functools.lru_cache(maxsize=None)
def _make_sc_aggregate(width: int):
    hw = width // NC
    mesh = plsc.VectorSubcoreMesh(core_axis_name="c", subcore_axis_name="s")

    def body(table, src_i, dst_i, out, tab_s, acc_s, src_v, dst_v, rows_v,
             gsem, ssem):
        c = lax.axis_index("c")
        s = lax.axis_index("s")
        r0 = s * RPT
        pltpu.sync_copy(table.at[c, pl.ds(r0, RPT)], tab_s.at[pl.ds(r0, RPT)])
        pltpu.sync_copy(table.at[c, pl.ds(r0, RPT)], acc_s.at[pl.ds(r0, RPT)])

        @pl.when(s == 0)
        def _init_tail():
            pltpu.sync_copy(table.at[c, pl.ds(TAIL0, TAIL)],
                            tab_s.at[pl.ds(TAIL0, TAIL)])
            pltpu.sync_copy(table.at[c, pl.ds(TAIL0, TAIL)],
                            acc_s.at[pl.ds(TAIL0, TAIL)])

        plsc.subcore_barrier()

        def outer(o, carry):
            pltpu.sync_copy(src_i.at[s, o], src_v)
            pltpu.sync_copy(dst_i.at[s, o], dst_v)

            def round_(r, carry2):
                gathers = []
                for b in range(NBUF):
                    j = r * NBUF + b
                    gathers.append(pltpu.async_copy(
                        tab_s.at[src_v.at[j]], rows_v.at[b], gsem.at[b]))
                scatters = []
                for b in range(NBUF):
                    j = r * NBUF + b
                    gathers[b].wait()
                    scatters.append(pltpu.async_copy(
                        rows_v.at[b], acc_s.at[dst_v.at[j]], ssem.at[b],
                        add=True))
                for b in range(NBUF):
                    scatters[b].wait()
                return carry2

            return lax.fori_loop(0, NROUND, round_, carry)

        lax.fori_loop(0, NOUTER, outer, 0)
        plsc.subcore_barrier()
        pltpu.sync_copy(acc_s.at[pl.ds(r0, RPT)], out.at[c, pl.ds(r0, RPT)])

        @pl.when(s == 0)
        def _out_tail():
            pltpu.sync_copy(acc_s.at[pl.ds(TAIL0, TAIL)],
                            out.at[c, pl.ds(TAIL0, TAIL)])

    return pl.kernel(
        body,
        out_type=jax.ShapeDtypeStruct((NC, N_NODES, hw), jnp.float32),
        mesh=mesh,
        scratch_types=[
            pltpu.VMEM_SHARED((N_NODES, hw), jnp.float32),
            pltpu.VMEM_SHARED((N_NODES, hw), jnp.float32),
            pltpu.VMEM((NINNER, CHUNK), jnp.int32),
            pltpu.VMEM((NINNER, CHUNK), jnp.int32),
            pltpu.VMEM((NBUF, CHUNK, hw), jnp.float32),
            pltpu.SemaphoreType.DMA((NBUF,)),
            pltpu.SemaphoreType.DMA((NBUF,)),
        ],
    )


def _tc_prelude(degp, x, w):
    def body(p_ref, x_ref, w_ref, hs_ref, dinv_ref):
        deg = p_ref[0, :, 0:1]
        dinv = lax.rsqrt(deg)
        h = jnp.dot(x_ref[...], w_ref[...], preferred_element_type=jnp.float32)
        hs = h * dinv
        hs_ref[0] = hs[:, : D_HID // 2]
        hs_ref[1] = hs[:, D_HID // 2:]
        dinv_ref[...] = dinv

    return pl.pallas_call(
        body,
        grid=(N_NODES // BR,),
        in_specs=[
            pl.BlockSpec((NC, BR, 8), lambda i: (0, i, 0)),
            pl.BlockSpec((BR, D_IN), lambda i: (i, 0)),
            pl.BlockSpec((D_IN, D_HID), lambda i: (0, 0)),
        ],
        out_specs=[
            pl.BlockSpec((NC, BR, D_HID // 2), lambda i: (0, i, 0)),
            pl.BlockSpec((BR, 1), lambda i: (i, 0)),
        ],
        out_shape=[
            jax.ShapeDtypeStruct((NC, N_NODES, D_HID // 2), jnp.float32),
            jax.ShapeDtypeStruct((N_NODES, 1), jnp.float32),
        ],
    )(degp, x, w)


def _tc_layer(p, dinv, w, b):
    wout = w.shape[1]

    def body(p_ref, dinv_ref, w_ref, b_ref, out_ref):
        agg = jnp.concatenate([p_ref[0], p_ref[1]], axis=1)
        t = agg * dinv_ref[...] + b_ref[...]
        r = jnp.maximum(t, 0.0)
        res = (
            jnp.dot(r, w_ref[...], preferred_element_type=jnp.float32)
            * dinv_ref[...]
        )
        out_ref[0] = res[:, : wout // 2]
        out_ref[1] = res[:, wout // 2:]

    return pl.pallas_call(
        body,
        grid=(N_NODES // BR,),
        in_specs=[
            pl.BlockSpec((NC, BR, D_HID // 2), lambda i: (0, i, 0)),
            pl.BlockSpec((BR, 1), lambda i: (i, 0)),
            pl.BlockSpec((D_HID, wout), lambda i: (0, 0)),
            pl.BlockSpec((1, D_HID), lambda i: (0, 0)),
        ],
        out_specs=pl.BlockSpec((NC, BR, wout // 2), lambda i: (0, i, 0)),
        out_shape=jax.ShapeDtypeStruct((NC, N_NODES, wout // 2), jnp.float32),
    )(p, dinv, w, b)


def _tc_final(p, dinv, b):
    def body(p_ref, dinv_ref, b_ref, out_ref):
        out_ref[...] = p_ref[0, :, 0:1] * dinv_ref[...] + b_ref[...]

    return pl.pallas_call(
        body,
        grid=(N_NODES // BR,),
        in_specs=[
            pl.BlockSpec((NC, BR, 8), lambda i: (0, i, 0)),
            pl.BlockSpec((BR, 1), lambda i: (i, 0)),
            pl.BlockSpec((1, 1), lambda i: (0, 0)),
        ],
        out_specs=pl.BlockSpec((BR, 1), lambda i: (i, 0)),
        out_shape=jax.ShapeDtypeStruct((N_NODES, 1), jnp.float32),
    )(p, dinv, b)


def kernel(x, edge_index, W_i, b_i, W_h, b_h, W_o, b_o):
    ei = edge_index.astype(jnp.int32)
    src4 = ei[0].reshape(NS, NOUTER, NINNER, CHUNK)
    dst4 = ei[1].reshape(NS, NOUTER, NINNER, CHUNK)
    onehot = jnp.zeros((NC, N_NODES, 8), jnp.float32).at[0, :, 0].set(1.0)

    agg16 = _make_sc_aggregate(16)
    agg64 = _make_sc_aggregate(D_HID)

    degp = agg16(onehot, src4, dst4)
    hs, dinv = _tc_prelude(degp, x, W_i)

    w_o16 = jnp.pad(W_o, ((0, 0), (0, 15)))
    weights = [W_h[i] for i in range(N_HID)] + [w_o16]
    biases = [b_i] + [b_h[i] for i in range(N_HID)]

    p = agg64(hs, src4, dst4)
    for i in range(N_HID + 1):
        nxt = _tc_layer(p, dinv, weights[i], biases[i].reshape(1, D_HID))
        p = (agg64 if i < N_HID else agg16)(nxt, src4, dst4)
    return _tc_final(p, dinv, b_o.reshape(1, 1))


# speedup vs baseline: 24.7302x; 1.0529x over previous
import functools

import jax
import jax.numpy as jnp
from jax import lax
from jax.experimental import pallas as pl
from jax.experimental.pallas import tpu as pltpu
from jax.experimental.pallas import tpu_sc as plsc

N_NODES = 10000
N_EDGES = 320000
D_IN = 128
D_HID = 64
N_HID = 6

NC = 2
NS = 16
NW = NC * NS
EPT = N_EDGES // NW
CHUNK = 80
NCHUNK = EPT // CHUNK
NOUTER = 5
NINNER = NCHUNK // NOUTER
NBUF = 5
NROUND = NINNER // NBUF
RPT = 624
TAIL = N_NODES - NS * RPT
TAIL0 = NS * RPT
BR = 1000


@functools.lru_cache(maxsize=None)
def _make_sc_aggregate(width: int):
    mesh = plsc.VectorSubcoreMesh(core_axis_name="c", subcore_axis_name="s")

    def body(table, src_i, dst_i, out, acc_s, src_v, dst_v, rows_v, gsem,
             ssem):
        c = lax.axis_index("c")
        s = lax.axis_index("s")
        wid = c * NS + s
        r0 = s * RPT
        pltpu.sync_copy(table.at[pl.ds(r0, RPT)], acc_s.at[pl.ds(r0, RPT)])

        @pl.when(s == 0)
        def _init_tail():
            pltpu.sync_copy(table.at[pl.ds(TAIL0, TAIL)],
                            acc_s.at[pl.ds(TAIL0, TAIL)])

        plsc.subcore_barrier()

        def outer(o, carry):
            pltpu.sync_copy(src_i.at[wid, o], src_v)
            pltpu.sync_copy(dst_i.at[wid, o], dst_v)

            def round_(r, carry2):
                gathers = []
                for b in range(NBUF):
                    j = r * NBUF + b
                    gathers.append(pltpu.async_copy(
                        table.at[src_v.at[j]], rows_v.at[b], gsem.at[b]))
                scatters = []
                for b in range(NBUF):
                    j = r * NBUF + b
                    gathers[b].wait()
                    scatters.append(pltpu.async_copy(
                        rows_v.at[b], acc_s.at[dst_v.at[j]], ssem.at[b],
                        add=True))
                for b in range(NBUF):
                    scatters[b].wait()
                return carry2

            return lax.fori_loop(0, NROUND, round_, carry)

        lax.fori_loop(0, NOUTER, outer, 0)
        plsc.subcore_barrier()
        pltpu.sync_copy(acc_s.at[pl.ds(r0, RPT)], out.at[c, pl.ds(r0, RPT)])

        @pl.when(s == 0)
        def _out_tail():
            pltpu.sync_copy(acc_s.at[pl.ds(TAIL0, TAIL)],
                            out.at[c, pl.ds(TAIL0, TAIL)])

    return pl.kernel(
        body,
        out_type=jax.ShapeDtypeStruct((NC, N_NODES, width), jnp.float32),
        mesh=mesh,
        scratch_types=[
            pltpu.VMEM_SHARED((N_NODES, width), jnp.float32),
            pltpu.VMEM((NINNER, CHUNK), jnp.int32),
            pltpu.VMEM((NINNER, CHUNK), jnp.int32),
            pltpu.VMEM((NBUF, CHUNK, width), jnp.float32),
            pltpu.SemaphoreType.DMA((NBUF,)),
            pltpu.SemaphoreType.DMA((NBUF,)),
        ],
        compiler_params=pltpu.CompilerParams(use_tc_tiling_on_sc=False),
    )


def _tc_prelude(degp, x, w):
    def body(p_ref, x_ref, w_ref, hs_ref, dinv_ref):
        deg = p_ref[0, :, 0:1] + p_ref[1, :, 0:1] - 1.0
        dinv = lax.rsqrt(deg)
        h = jnp.dot(x_ref[...], w_ref[...], preferred_element_type=jnp.float32)
        hs_ref[...] = h * dinv
        dinv_ref[...] = dinv

    return pl.pallas_call(
        body,
        grid=(N_NODES // BR,),
        in_specs=[
            pl.BlockSpec((NC, BR, 16), lambda i: (0, i, 0)),
            pl.BlockSpec((BR, D_IN), lambda i: (i, 0)),
            pl.BlockSpec((D_IN, D_HID), lambda i: (0, 0)),
        ],
        out_specs=[
            pl.BlockSpec((BR, D_HID), lambda i: (i, 0)),
            pl.BlockSpec((BR, 1), lambda i: (i, 0)),
        ],
        out_shape=[
            jax.ShapeDtypeStruct((N_NODES, D_HID), jnp.float32),
            jax.ShapeDtypeStruct((N_NODES, 1), jnp.float32),
        ],
    )(degp, x, w)


def _tc_layer(p, hs, dinv, w, b):
    wout = w.shape[1]

    def body(p_ref, hs_ref, dinv_ref, w_ref, b_ref, out_ref):
        agg = p_ref[0] + p_ref[1] - hs_ref[...]
        t = agg * dinv_ref[...] + b_ref[...]
        r = jnp.maximum(t, 0.0)
        out_ref[...] = (
            jnp.dot(r, w_ref[...], preferred_element_type=jnp.float32)
            * dinv_ref[...]
        )

    return pl.pallas_call(
        body,
        grid=(N_NODES // BR,),
        in_specs=[
            pl.BlockSpec((NC, BR, D_HID), lambda i: (0, i, 0)),
            pl.BlockSpec((BR, D_HID), lambda i: (i, 0)),
            pl.BlockSpec((BR, 1), lambda i: (i, 0)),
            pl.BlockSpec((D_HID, wout), lambda i: (0, 0)),
            pl.BlockSpec((1, D_HID), lambda i: (0, 0)),
        ],
        out_specs=pl.BlockSpec((BR, wout), lambda i: (i, 0)),
        out_shape=jax.ShapeDtypeStruct((N_NODES, wout), jnp.float32),
    )(p, hs, dinv, w, b)


def _tc_final(p, z, dinv, b):
    def body(p_ref, z_ref, dinv_ref, b_ref, out_ref):
        agg = p_ref[0] + p_ref[1] - z_ref[...]
        out_ref[...] = agg[:, 0:1] * dinv_ref[...] + b_ref[...]

    return pl.pallas_call(
        body,
        grid=(N_NODES // BR,),
        in_specs=[
            pl.BlockSpec((NC, BR, 16), lambda i: (0, i, 0)),
            pl.BlockSpec((BR, 16), lambda i: (i, 0)),
            pl.BlockSpec((BR, 1), lambda i: (i, 0)),
            pl.BlockSpec((1, 1), lambda i: (0, 0)),
        ],
        out_specs=pl.BlockSpec((BR, 1), lambda i: (i, 0)),
        out_shape=jax.ShapeDtypeStruct((N_NODES, 1), jnp.float32),
    )(p, z, dinv, b)


def kernel(x, edge_index, W_i, b_i, W_h, b_h, W_o, b_o):
    ei = edge_index.astype(jnp.int32)
    src4 = ei[0].reshape(NW, NOUTER, NINNER, CHUNK)
    dst4 = ei[1].reshape(NW, NOUTER, NINNER, CHUNK)
    onehot = jnp.zeros((N_NODES, 16), jnp.float32).at[:, 0].set(1.0)

    agg16 = _make_sc_aggregate(16)
    agg64 = _make_sc_aggregate(D_HID)

    degp = agg16(onehot, src4, dst4)
    hs, dinv = _tc_prelude(degp, x, W_i)

    w_o16 = jnp.pad(W_o, ((0, 0), (0, 15)))
    weights = [W_h[i] for i in range(N_HID)] + [w_o16]
    biases = [b_i] + [b_h[i] for i in range(N_HID)]

    p = agg64(hs, src4, dst4)
    for i in range(N_HID + 1):
        nxt = _tc_layer(p, hs, dinv, weights[i], biases[i].reshape(1, D_HID))
        hs = nxt
        p = (agg64 if i < N_HID else agg16)(nxt, src4, dst4)
    return _tc_final(p, hs, dinv, b_o.reshape(1, 1))


# speedup vs baseline: 30.3187x; 1.2260x over previous
import functools

import jax
import jax.numpy as jnp
from jax import lax
from jax.experimental import pallas as pl
from jax.experimental.pallas import tpu as pltpu
from jax.experimental.pallas import tpu_sc as plsc

N_NODES = 10000
N_EDGES = 320000
D_IN = 128
D_HID = 64
N_HID = 6

NC = 2
NS = 16
NW = NC * NS
EPT = N_EDGES // NW
CHUNK = 80
NCHUNK = EPT // CHUNK
NBUF = 5
NROUND = NCHUNK // NBUF
RPT = 624
TAIL = N_NODES - NS * RPT
TAIL0 = NS * RPT
BR = 1000


@functools.lru_cache(maxsize=None)
def _make_sc_aggregate(width: int):
    mesh = plsc.VectorSubcoreMesh(core_axis_name="c", subcore_axis_name="s")

    def body(table, src_i, dst_i, out, acc_s, src_v, dst_v, rows_v, gsem,
             ssem):
        c = lax.axis_index("c")
        s = lax.axis_index("s")
        wid = c * NS + s
        r0 = s * RPT
        pltpu.sync_copy(table.at[pl.ds(r0, RPT)], acc_s.at[pl.ds(r0, RPT)])

        @pl.when(s == 0)
        def _init_tail():
            pltpu.sync_copy(table.at[pl.ds(TAIL0, TAIL)],
                            acc_s.at[pl.ds(TAIL0, TAIL)])

        pltpu.sync_copy(src_i.at[wid], src_v)
        pltpu.sync_copy(dst_i.at[wid], dst_v)
        for b in range(NBUF):
            pltpu.async_copy(table.at[src_v.at[b]], rows_v.at[b],
                             gsem.at[b])
        plsc.subcore_barrier()

        def gwait(j, b):
            return pltpu.make_async_copy(
                table.at[src_v.at[j]], rows_v.at[b], gsem.at[b]).wait()

        def round_(r, carry):
            scatters = []
            for b in range(NBUF):
                j = r * NBUF + b
                gwait(j, b)
                scatters.append(pltpu.async_copy(
                    rows_v.at[b], acc_s.at[dst_v.at[j]], ssem.at[b],
                    add=True))
            for b in range(NBUF):
                scatters[b].wait()
                jn = (r + 1) * NBUF + b
                pltpu.async_copy(table.at[src_v.at[jn]], rows_v.at[b],
                                 gsem.at[b])
            return carry

        lax.fori_loop(0, NROUND - 1, round_, 0)
        last = []
        for b in range(NBUF):
            j = (NROUND - 1) * NBUF + b
            gwait(j, b)
            last.append(pltpu.async_copy(
                rows_v.at[b], acc_s.at[dst_v.at[j]], ssem.at[b], add=True))
        for d in last:
            d.wait()
        plsc.subcore_barrier()
        pltpu.sync_copy(acc_s.at[pl.ds(r0, RPT)], out.at[c, pl.ds(r0, RPT)])

        @pl.when(s == 0)
        def _out_tail():
            pltpu.sync_copy(acc_s.at[pl.ds(TAIL0, TAIL)],
                            out.at[c, pl.ds(TAIL0, TAIL)])

    return pl.kernel(
        body,
        out_type=jax.ShapeDtypeStruct((NC, N_NODES, width), jnp.float32),
        mesh=mesh,
        scratch_types=[
            pltpu.VMEM_SHARED((N_NODES, width), jnp.float32),
            pltpu.VMEM((NCHUNK, CHUNK), jnp.int32),
            pltpu.VMEM((NCHUNK, CHUNK), jnp.int32),
            pltpu.VMEM((NBUF, CHUNK, width), jnp.float32),
            pltpu.SemaphoreType.DMA((NBUF,)),
            pltpu.SemaphoreType.DMA((NBUF,)),
        ],
        compiler_params=pltpu.CompilerParams(use_tc_tiling_on_sc=False),
    )


def _tc_matmul0(x, w):
    def body(x_ref, w_ref, h_ref):
        h_ref[...] = jnp.dot(x_ref[...], w_ref[...],
                             preferred_element_type=jnp.float32)

    return pl.pallas_call(
        body,
        grid=(N_NODES // BR,),
        in_specs=[
            pl.BlockSpec((BR, D_IN), lambda i: (i, 0)),
            pl.BlockSpec((D_IN, D_HID), lambda i: (0, 0)),
        ],
        out_specs=pl.BlockSpec((BR, D_HID), lambda i: (i, 0)),
        out_shape=jax.ShapeDtypeStruct((N_NODES, D_HID), jnp.float32),
    )(x, w)


def _tc_scale0(degp, h):
    def body(p_ref, h_ref, hs_ref, dinv_ref):
        deg = p_ref[0, :, 0:1] + p_ref[1, :, 0:1] - 1.0
        dinv = lax.rsqrt(deg)
        hs_ref[...] = h_ref[...] * dinv
        dinv_ref[...] = dinv

    return pl.pallas_call(
        body,
        grid=(N_NODES // BR,),
        in_specs=[
            pl.BlockSpec((NC, BR, 16), lambda i: (0, i, 0)),
            pl.BlockSpec((BR, D_HID), lambda i: (i, 0)),
        ],
        out_specs=[
            pl.BlockSpec((BR, D_HID), lambda i: (i, 0)),
            pl.BlockSpec((BR, 1), lambda i: (i, 0)),
        ],
        out_shape=[
            jax.ShapeDtypeStruct((N_NODES, D_HID), jnp.float32),
            jax.ShapeDtypeStruct((N_NODES, 1), jnp.float32),
        ],
    )(degp, h)


def _tc_layer(p, hs, dinv, w, b):
    wout = w.shape[1]

    def body(p_ref, hs_ref, dinv_ref, w_ref, b_ref, out_ref):
        agg = p_ref[0] + p_ref[1] - hs_ref[...]
        t = agg * dinv_ref[...] + b_ref[...]
        r = jnp.maximum(t, 0.0)
        out_ref[...] = (
            jnp.dot(r, w_ref[...], preferred_element_type=jnp.float32)
            * dinv_ref[...]
        )

    return pl.pallas_call(
        body,
        grid=(N_NODES // BR,),
        in_specs=[
            pl.BlockSpec((NC, BR, D_HID), lambda i: (0, i, 0)),
            pl.BlockSpec((BR, D_HID), lambda i: (i, 0)),
            pl.BlockSpec((BR, 1), lambda i: (i, 0)),
            pl.BlockSpec((D_HID, wout), lambda i: (0, 0)),
            pl.BlockSpec((1, D_HID), lambda i: (0, 0)),
        ],
        out_specs=pl.BlockSpec((BR, wout), lambda i: (i, 0)),
        out_shape=jax.ShapeDtypeStruct((N_NODES, wout), jnp.float32),
    )(p, hs, dinv, w, b)


def _tc_final(p, z, dinv, b):
    def body(p_ref, z_ref, dinv_ref, b_ref, out_ref):
        agg = p_ref[0] + p_ref[1] - z_ref[...]
        out_ref[...] = agg[:, 0:1] * dinv_ref[...] + b_ref[...]

    return pl.pallas_call(
        body,
        grid=(N_NODES // BR,),
        in_specs=[
            pl.BlockSpec((NC, BR, 16), lambda i: (0, i, 0)),
            pl.BlockSpec((BR, 16), lambda i: (i, 0)),
            pl.BlockSpec((BR, 1), lambda i: (i, 0)),
            pl.BlockSpec((1, 1), lambda i: (0, 0)),
        ],
        out_specs=pl.BlockSpec((BR, 1), lambda i: (i, 0)),
        out_shape=jax.ShapeDtypeStruct((N_NODES, 1), jnp.float32),
    )(p, z, dinv, b)


def kernel(x, edge_index, W_i, b_i, W_h, b_h, W_o, b_o):
    ei = edge_index.astype(jnp.int32)
    src4 = ei[0].reshape(NW, NCHUNK, CHUNK)
    dst4 = ei[1].reshape(NW, NCHUNK, CHUNK)
    onehot = jnp.zeros((N_NODES, 16), jnp.float32).at[:, 0].set(1.0)

    agg16 = _make_sc_aggregate(16)
    agg64 = _make_sc_aggregate(D_HID)

    degp = agg16(onehot, src4, dst4)
    h0 = _tc_matmul0(x, W_i)
    hs, dinv = _tc_scale0(degp, h0)

    w_o16 = jnp.pad(W_o, ((0, 0), (0, 15)))
    weights = [W_h[i] for i in range(N_HID)] + [w_o16]
    biases = [b_i] + [b_h[i] for i in range(N_HID)]

    p = agg64(hs, src4, dst4)
    for i in range(N_HID + 1):
        nxt = _tc_layer(p, hs, dinv, weights[i], biases[i].reshape(1, D_HID))
        hs = nxt
        p = (agg64 if i < N_HID else agg16)(nxt, src4, dst4)
    return _tc_final(p, hs, dinv, b_o.reshape(1, 1))


# speedup vs baseline: 30.6811x; 1.0120x over previous
import functools

import jax
import jax.numpy as jnp
from jax import lax
from jax.experimental import pallas as pl
from jax.experimental.pallas import tpu as pltpu
from jax.experimental.pallas import tpu_sc as plsc

N_NODES = 10000
N_EDGES = 320000
D_IN = 128
D_HID = 64
N_HID = 6

NC = 2
NS = 16
NW = NC * NS
EPT = N_EDGES // NW
CHUNK = 80
NCHUNK = EPT // CHUNK
NBUF = 5
NROUND = NCHUNK // NBUF
RPT = 624
TAIL = N_NODES - NS * RPT
TAIL0 = NS * RPT
BR = 1000


@functools.lru_cache(maxsize=None)
def _make_sc_aggregate(width: int):
    mesh = plsc.VectorSubcoreMesh(core_axis_name="c", subcore_axis_name="s")

    def body(table, src_i, dst_i, out, acc_s, src_v, dst_v, rows_v, gsem,
             ssem):
        c = lax.axis_index("c")
        s = lax.axis_index("s")
        wid = c * NS + s
        r0 = s * RPT
        pltpu.sync_copy(table.at[pl.ds(r0, RPT)], acc_s.at[pl.ds(r0, RPT)])

        @pl.when(s == 0)
        def _init_tail():
            pltpu.sync_copy(table.at[pl.ds(TAIL0, TAIL)],
                            acc_s.at[pl.ds(TAIL0, TAIL)])

        pltpu.sync_copy(src_i.at[wid], src_v)
        pltpu.sync_copy(dst_i.at[wid], dst_v)
        for b in range(NBUF):
            pltpu.async_copy(table.at[src_v.at[b]], rows_v.at[b],
                             gsem.at[b])
        plsc.subcore_barrier()

        def gwait(j, b):
            return pltpu.make_async_copy(
                table.at[src_v.at[j]], rows_v.at[b], gsem.at[b]).wait()

        def round_(r, carry):
            scatters = []
            for b in range(NBUF):
                j = r * NBUF + b
                gwait(j, b)
                scatters.append(pltpu.async_copy(
                    rows_v.at[b], acc_s.at[dst_v.at[j]], ssem.at[b],
                    add=True))
            for b in range(NBUF):
                scatters[b].wait()
                jn = (r + 1) * NBUF + b
                pltpu.async_copy(table.at[src_v.at[jn]], rows_v.at[b],
                                 gsem.at[b])
            return carry

        lax.fori_loop(0, NROUND - 1, round_, 0)
        last = []
        for b in range(NBUF):
            j = (NROUND - 1) * NBUF + b
            gwait(j, b)
            last.append(pltpu.async_copy(
                rows_v.at[b], acc_s.at[dst_v.at[j]], ssem.at[b], add=True))
        for d in last:
            d.wait()
        plsc.subcore_barrier()
        pltpu.sync_copy(acc_s.at[pl.ds(r0, RPT)], out.at[c, pl.ds(r0, RPT)])

        @pl.when(s == 0)
        def _out_tail():
            pltpu.sync_copy(acc_s.at[pl.ds(TAIL0, TAIL)],
                            out.at[c, pl.ds(TAIL0, TAIL)])

    return pl.kernel(
        body,
        out_type=jax.ShapeDtypeStruct((NC, N_NODES, width), jnp.float32),
        mesh=mesh,
        scratch_types=[
            pltpu.VMEM_SHARED((N_NODES, width), jnp.float32),
            pltpu.VMEM((NCHUNK, CHUNK), jnp.int32),
            pltpu.VMEM((NCHUNK, CHUNK), jnp.int32),
            pltpu.VMEM((NBUF, CHUNK, width), jnp.float32),
            pltpu.SemaphoreType.DMA((NBUF,)),
            pltpu.SemaphoreType.DMA((NBUF,)),
        ],
        compiler_params=pltpu.CompilerParams(use_tc_tiling_on_sc=False),
    )


def _tc_matmul0(x, w):
    def body(x_ref, w_ref, h_ref):
        h_ref[...] = jnp.dot(x_ref[...], w_ref[...],
                             preferred_element_type=jnp.float32)

    return pl.pallas_call(
        body,
        grid=(N_NODES // BR,),
        in_specs=[
            pl.BlockSpec((BR, D_IN), lambda i: (i, 0)),
            pl.BlockSpec((D_IN, D_HID), lambda i: (0, 0)),
        ],
        out_specs=pl.BlockSpec((BR, D_HID), lambda i: (i, 0)),
        out_shape=jax.ShapeDtypeStruct((N_NODES, D_HID), jnp.float32),
    )(x, w)


def _tc_scale0(degp, h):
    def body(p_ref, h_ref, hs_ref, dinv_ref):
        deg = p_ref[0, :, 0:1] + p_ref[1, :, 0:1] - 1.0
        dinv = lax.rsqrt(deg)
        hs_ref[...] = h_ref[...] * dinv
        dinv_ref[...] = dinv

    return pl.pallas_call(
        body,
        grid=(N_NODES // BR,),
        in_specs=[
            pl.BlockSpec((NC, BR, 16), lambda i: (0, i, 0)),
            pl.BlockSpec((BR, D_HID), lambda i: (i, 0)),
        ],
        out_specs=[
            pl.BlockSpec((BR, D_HID), lambda i: (i, 0)),
            pl.BlockSpec((BR, 1), lambda i: (i, 0)),
        ],
        out_shape=[
            jax.ShapeDtypeStruct((N_NODES, D_HID), jnp.float32),
            jax.ShapeDtypeStruct((N_NODES, 1), jnp.float32),
        ],
    )(degp, h)


def _tc_layer(p, hs, dinv, w, b):
    wout = w.shape[1]

    def body(p_ref, hs_ref, dinv_ref, w_ref, b_ref, out_ref):
        agg = p_ref[0] + p_ref[1] - hs_ref[...]
        t = agg * dinv_ref[...] + b_ref[...]
        r = jnp.maximum(t, 0.0)
        out_ref[...] = (
            jnp.dot(r, w_ref[...], preferred_element_type=jnp.float32)
            * dinv_ref[...]
        )

    return pl.pallas_call(
        body,
        grid=(N_NODES // BR,),
        in_specs=[
            pl.BlockSpec((NC, BR, D_HID), lambda i: (0, i, 0)),
            pl.BlockSpec((BR, D_HID), lambda i: (i, 0)),
            pl.BlockSpec((BR, 1), lambda i: (i, 0)),
            pl.BlockSpec((D_HID, wout), lambda i: (0, 0)),
            pl.BlockSpec((1, D_HID), lambda i: (0, 0)),
        ],
        out_specs=pl.BlockSpec((BR, wout), lambda i: (i, 0)),
        out_shape=jax.ShapeDtypeStruct((N_NODES, wout), jnp.float32),
    )(p, hs, dinv, w, b)


def _tc_final(p, z, dinv, b):
    def body(p_ref, z_ref, dinv_ref, b_ref, out_ref):
        agg = p_ref[0] + p_ref[1] - z_ref[...]
        out_ref[...] = agg[:, 0:1] * dinv_ref[...] + b_ref[...]

    return pl.pallas_call(
        body,
        grid=(N_NODES // BR,),
        in_specs=[
            pl.BlockSpec((NC, BR, 16), lambda i: (0, i, 0)),
            pl.BlockSpec((BR, 16), lambda i: (i, 0)),
            pl.BlockSpec((BR, 1), lambda i: (i, 0)),
            pl.BlockSpec((1, 1), lambda i: (0, 0)),
        ],
        out_specs=pl.BlockSpec((BR, 1), lambda i: (i, 0)),
        out_shape=jax.ShapeDtypeStruct((N_NODES, 1), jnp.float32),
    )(p, z, dinv, b)


def kernel(x, edge_index, W_i, b_i, W_h, b_h, W_o, b_o):
    ei = edge_index.astype(jnp.int32)
    src4 = ei[0].reshape(NW, NCHUNK, CHUNK)
    dst4 = ei[1].reshape(NW, NCHUNK, CHUNK)
    onehot = jnp.zeros((N_NODES, 16), jnp.float32).at[:, 0].set(1.0)

    agg16 = _make_sc_aggregate(16)
    agg64 = _make_sc_aggregate(D_HID)

    degp = agg16(onehot, src4, dst4)
    h0 = _tc_matmul0(x, W_i)
    hs, dinv = _tc_scale0(degp, h0)

    w_o16 = jnp.pad(W_o, ((0, 0), (0, 15)))

    p = agg64(hs, src4, dst4)

    w_stack = W_h
    b_stack = jnp.concatenate(
        [b_i.reshape(1, D_HID)] + [b_h[i].reshape(1, D_HID)
                                   for i in range(N_HID - 1)], axis=0)

    def step(carry, wb):
        p_c, hs_c = carry
        w, b = wb
        nxt = _tc_layer(p_c, hs_c, dinv, w, b.reshape(1, D_HID))
        return (agg64(nxt, src4, dst4), nxt), None

    (p, hs), _ = lax.scan(step, (p, hs), (w_stack, b_stack))

    nxt = _tc_layer(p, hs, dinv, w_o16, b_h[N_HID - 1].reshape(1, D_HID))
    p = agg16(nxt, src4, dst4)
    return _tc_final(p, nxt, dinv, b_o.reshape(1, 1))


# speedup vs baseline: 31.3124x; 1.0206x over previous
import functools

import jax
import jax.numpy as jnp
from jax import lax
from jax.experimental import pallas as pl
from jax.experimental.pallas import tpu as pltpu
from jax.experimental.pallas import tpu_sc as plsc

N_NODES = 10000
N_EDGES = 320000
D_IN = 128
D_HID = 64
N_HID = 6

NC = 2
NS = 16
NW = NC * NS
EPT = N_EDGES // NW
CHUNK = 80
NCHUNK = EPT // CHUNK
NBUF = 5
NROUND = NCHUNK // NBUF
RPT = 624
TAIL = N_NODES - NS * RPT
TAIL0 = NS * RPT
BR = 1000


@functools.lru_cache(maxsize=None)
def _make_sc_aggregate(width: int):
    mesh = plsc.VectorSubcoreMesh(core_axis_name="c", subcore_axis_name="s")

    def body(table, src_i, dst_i, out, acc_s, src_v, dst_v, rows_v, gsem,
             ssem):
        c = lax.axis_index("c")
        s = lax.axis_index("s")
        wid = c * NS + s
        r0 = s * RPT
        pltpu.sync_copy(table.at[pl.ds(r0, RPT)], acc_s.at[pl.ds(r0, RPT)])

        @pl.when(s == 0)
        def _init_tail():
            pltpu.sync_copy(table.at[pl.ds(TAIL0, TAIL)],
                            acc_s.at[pl.ds(TAIL0, TAIL)])

        pltpu.sync_copy(src_i.at[wid], src_v)
        pltpu.sync_copy(dst_i.at[wid], dst_v)
        for b in range(NBUF):
            pltpu.async_copy(table.at[src_v.at[b]], rows_v.at[b],
                             gsem.at[b])
        plsc.subcore_barrier()

        def gwait(j, b):
            return pltpu.make_async_copy(
                table.at[src_v.at[j]], rows_v.at[b], gsem.at[b]).wait()

        def round_(r, carry):
            scatters = []
            for b in range(NBUF):
                j = r * NBUF + b
                gwait(j, b)
                scatters.append(pltpu.async_copy(
                    rows_v.at[b], acc_s.at[dst_v.at[j]], ssem.at[b],
                    add=True))
            for b in range(NBUF):
                scatters[b].wait()
                jn = (r + 1) * NBUF + b
                pltpu.async_copy(table.at[src_v.at[jn]], rows_v.at[b],
                                 gsem.at[b])
            return carry

        lax.fori_loop(0, NROUND - 1, round_, 0)
        last = []
        for b in range(NBUF):
            j = (NROUND - 1) * NBUF + b
            gwait(j, b)
            last.append(pltpu.async_copy(
                rows_v.at[b], acc_s.at[dst_v.at[j]], ssem.at[b], add=True))
        for d in last:
            d.wait()
        plsc.subcore_barrier()
        pltpu.sync_copy(acc_s.at[pl.ds(r0, RPT)], out.at[c, pl.ds(r0, RPT)])

        @pl.when(s == 0)
        def _out_tail():
            pltpu.sync_copy(acc_s.at[pl.ds(TAIL0, TAIL)],
                            out.at[c, pl.ds(TAIL0, TAIL)])

    return pl.kernel(
        body,
        out_type=jax.ShapeDtypeStruct((NC, N_NODES, width), jnp.float32),
        mesh=mesh,
        scratch_types=[
            pltpu.VMEM_SHARED((N_NODES, width), jnp.float32),
            pltpu.VMEM((NCHUNK, CHUNK), jnp.int32),
            pltpu.VMEM((NCHUNK, CHUNK), jnp.int32),
            pltpu.VMEM((NBUF, CHUNK, width), jnp.float32),
            pltpu.SemaphoreType.DMA((NBUF,)),
            pltpu.SemaphoreType.DMA((NBUF,)),
        ],
        compiler_params=pltpu.CompilerParams(use_tc_tiling_on_sc=False),
    )


def _make_sc_degree():
    width = 16
    mesh = plsc.VectorSubcoreMesh(core_axis_name="c", subcore_axis_name="s")

    def body(table, ones_rows, dst_i, out, acc_s, dst_v, rows_v, ssem):
        c = lax.axis_index("c")
        s = lax.axis_index("s")
        wid = c * NS + s
        r0 = s * RPT
        pltpu.sync_copy(table.at[pl.ds(r0, RPT)], acc_s.at[pl.ds(r0, RPT)])

        @pl.when(s == 0)
        def _init_tail():
            pltpu.sync_copy(table.at[pl.ds(TAIL0, TAIL)],
                            acc_s.at[pl.ds(TAIL0, TAIL)])

        pltpu.sync_copy(dst_i.at[wid], dst_v)
        pltpu.sync_copy(ones_rows, rows_v)
        plsc.subcore_barrier()

        def round_(r, carry):
            scatters = []
            for b in range(NBUF):
                j = r * NBUF + b
                scatters.append(pltpu.async_copy(
                    rows_v, acc_s.at[dst_v.at[j]], ssem.at[b], add=True))
            for d in scatters:
                d.wait()
            return carry

        lax.fori_loop(0, NROUND, round_, 0)
        plsc.subcore_barrier()
        pltpu.sync_copy(acc_s.at[pl.ds(r0, RPT)], out.at[c, pl.ds(r0, RPT)])

        @pl.when(s == 0)
        def _out_tail():
            pltpu.sync_copy(acc_s.at[pl.ds(TAIL0, TAIL)],
                            out.at[c, pl.ds(TAIL0, TAIL)])

    return pl.kernel(
        body,
        out_type=jax.ShapeDtypeStruct((NC, N_NODES, width), jnp.float32),
        mesh=mesh,
        scratch_types=[
            pltpu.VMEM_SHARED((N_NODES, width), jnp.float32),
            pltpu.VMEM((NCHUNK, CHUNK), jnp.int32),
            pltpu.VMEM((CHUNK, width), jnp.float32),
            pltpu.SemaphoreType.DMA((NBUF,)),
        ],
        compiler_params=pltpu.CompilerParams(use_tc_tiling_on_sc=False),
    )


def _tc_matmul0(x, w):
    def body(x_ref, w_ref, h_ref):
        h_ref[...] = jnp.dot(x_ref[...], w_ref[...],
                             preferred_element_type=jnp.float32)

    return pl.pallas_call(
        body,
        grid=(N_NODES // BR,),
        in_specs=[
            pl.BlockSpec((BR, D_IN), lambda i: (i, 0)),
            pl.BlockSpec((D_IN, D_HID), lambda i: (0, 0)),
        ],
        out_specs=pl.BlockSpec((BR, D_HID), lambda i: (i, 0)),
        out_shape=jax.ShapeDtypeStruct((N_NODES, D_HID), jnp.float32),
    )(x, w)


def _tc_scale0(degp, h):
    def body(p_ref, h_ref, hs_ref, dinv_ref):
        deg = p_ref[0, :, 0:1] + p_ref[1, :, 0:1] - 1.0
        dinv = lax.rsqrt(deg)
        hs_ref[...] = h_ref[...] * dinv
        dinv_ref[...] = dinv

    return pl.pallas_call(
        body,
        grid=(N_NODES // BR,),
        in_specs=[
            pl.BlockSpec((NC, BR, 16), lambda i: (0, i, 0)),
            pl.BlockSpec((BR, D_HID), lambda i: (i, 0)),
        ],
        out_specs=[
            pl.BlockSpec((BR, D_HID), lambda i: (i, 0)),
            pl.BlockSpec((BR, 1), lambda i: (i, 0)),
        ],
        out_shape=[
            jax.ShapeDtypeStruct((N_NODES, D_HID), jnp.float32),
            jax.ShapeDtypeStruct((N_NODES, 1), jnp.float32),
        ],
    )(degp, h)


def _tc_layer(p, hs, dinv, w, b):
    wout = w.shape[1]

    def body(p_ref, hs_ref, dinv_ref, w_ref, b_ref, out_ref):
        agg = p_ref[0] + p_ref[1] - hs_ref[...]
        t = agg * dinv_ref[...] + b_ref[...]
        r = jnp.maximum(t, 0.0)
        out_ref[...] = (
            jnp.dot(r, w_ref[...], preferred_element_type=jnp.float32)
            * dinv_ref[...]
        )

    return pl.pallas_call(
        body,
        grid=(N_NODES // BR,),
        in_specs=[
            pl.BlockSpec((NC, BR, D_HID), lambda i: (0, i, 0)),
            pl.BlockSpec((BR, D_HID), lambda i: (i, 0)),
            pl.BlockSpec((BR, 1), lambda i: (i, 0)),
            pl.BlockSpec((D_HID, wout), lambda i: (0, 0)),
            pl.BlockSpec((1, D_HID), lambda i: (0, 0)),
        ],
        out_specs=pl.BlockSpec((BR, wout), lambda i: (i, 0)),
        out_shape=jax.ShapeDtypeStruct((N_NODES, wout), jnp.float32),
    )(p, hs, dinv, w, b)


def _tc_final(p, z, dinv, b):
    def body(p_ref, z_ref, dinv_ref, b_ref, out_ref):
        agg = p_ref[0] + p_ref[1] - z_ref[...]
        out_ref[...] = agg[:, 0:1] * dinv_ref[...] + b_ref[...]

    return pl.pallas_call(
        body,
        grid=(N_NODES // BR,),
        in_specs=[
            pl.BlockSpec((NC, BR, 16), lambda i: (0, i, 0)),
            pl.BlockSpec((BR, 16), lambda i: (i, 0)),
            pl.BlockSpec((BR, 1), lambda i: (i, 0)),
            pl.BlockSpec((1, 1), lambda i: (0, 0)),
        ],
        out_specs=pl.BlockSpec((BR, 1), lambda i: (i, 0)),
        out_shape=jax.ShapeDtypeStruct((N_NODES, 1), jnp.float32),
    )(p, z, dinv, b)


def kernel(x, edge_index, W_i, b_i, W_h, b_h, W_o, b_o):
    ei = edge_index.astype(jnp.int32)
    src4 = ei[0].reshape(NW, NCHUNK, CHUNK)
    dst4 = ei[1].reshape(NW, NCHUNK, CHUNK)
    onehot = jnp.zeros((N_NODES, 16), jnp.float32).at[:, 0].set(1.0)
    ones_rows = jnp.zeros((CHUNK, 16), jnp.float32).at[:, 0].set(1.0)

    agg16 = _make_sc_aggregate(16)
    agg64 = _make_sc_aggregate(D_HID)

    degp = _make_sc_degree()(onehot, ones_rows, dst4)
    h0 = _tc_matmul0(x, W_i)
    hs, dinv = _tc_scale0(degp, h0)

    w_o16 = jnp.pad(W_o, ((0, 0), (0, 15)))

    p = agg64(hs, src4, dst4)

    w_stack = W_h
    b_stack = jnp.concatenate(
        [b_i.reshape(1, D_HID)] + [b_h[i].reshape(1, D_HID)
                                   for i in range(N_HID - 1)], axis=0)

    def step(carry, wb):
        p_c, hs_c = carry
        w, b = wb
        nxt = _tc_layer(p_c, hs_c, dinv, w, b.reshape(1, D_HID))
        return (agg64(nxt, src4, dst4), nxt), None

    (p, hs), _ = lax.scan(step, (p, hs), (w_stack, b_stack))

    nxt = _tc_layer(p, hs, dinv, w_o16, b_h[N_HID - 1].reshape(1, D_HID))
    p = agg16(nxt, src4, dst4)
    return _tc_final(p, nxt, dinv, b_o.reshape(1, 1))
